# Initial kernel scaffold; baseline (speedup 1.0000x reference)
import jax
import jax.numpy as jnp
from jax.experimental import pallas as pl


def kernel(point_features, point_coords, g_W1, g_b1, g_W2, g_b2, c_W1, c_b1, c_W2, c_b2, d_W1, d_b1, d_W2, d_b2, m_W1, m_b1, m_W2, m_b2, ln_g, ln_b, q_A, q_B, q_s, k_A, k_B, k_s, v_A, v_B, v_s):
    raise NotImplementedError("write your pallas kernel here")


# speedup vs baseline: 35.8631x; 35.8631x over previous
import functools

import jax
import jax.numpy as jnp
from jax import lax
from jax.experimental import pallas as pl
from jax.experimental.pallas import tpu as pltpu
from jax.experimental.pallas import tpu_sc as plsc

_B, _N, _D = 4, 4096, 768
_C = 256
_LORA_SCALING = 32.0 / 16.0


def _fps_safety_body(x_ref, y_ref, z_ref, ci_ref, ssg_ref, ssd_ref, ssc_ref):
    x = x_ref[...]
    y = y_ref[...]
    z = z_ref[...]
    lane = lax.broadcasted_iota(jnp.int32, (_B, _N), 1)
    col_c = lax.broadcasted_iota(jnp.int32, (_B, _C), 1)

    def extract(v, last):
        return jnp.sum(jnp.where(lane == last, v, 0.0), axis=1, keepdims=True)

    def body(i, st):
        dist, ci, ccx, ccy, ccz, last = st
        px = extract(x, last)
        py = extract(y, last)
        pz = extract(z, last)
        ccx = jnp.where(col_c == i - 1, px, ccx)
        ccy = jnp.where(col_c == i - 1, py, ccy)
        ccz = jnp.where(col_c == i - 1, pz, ccz)
        dx = x - px
        dy = y - py
        dz = z - pz
        d = jnp.sqrt(dx * dx + dy * dy + dz * dz)
        dist = jnp.minimum(dist, d)
        m = jnp.max(dist, axis=1, keepdims=True)
        nxt = jnp.min(jnp.where(dist == m, lane, _N), axis=1, keepdims=True)
        dist = jnp.where(lane == nxt, 0.0, dist)
        ci = jnp.where(col_c == i, nxt, ci)
        return dist, ci, ccx, ccy, ccz, nxt

    dist0 = jnp.full((_B, _N), jnp.inf, jnp.float32)
    ci0 = jnp.zeros((_B, _C), jnp.int32)
    cc0 = jnp.zeros((_B, _C), jnp.float32)
    last0 = jnp.zeros((_B, 1), jnp.int32)
    _, ci, ccx, ccy, ccz, last = lax.fori_loop(
        1, _C, body, (dist0, ci0, cc0, cc0, cc0, last0))
    px = extract(x, last)
    py = extract(y, last)
    pz = extract(z, last)
    ccx = jnp.where(col_c == _C - 1, px, ccx)
    ccy = jnp.where(col_c == _C - 1, py, ccy)
    ccz = jnp.where(col_c == _C - 1, pz, ccz)

    meanz = jnp.mean(z, axis=1, keepdims=True)
    ssg_ref[...] = 1.0 + jax.nn.sigmoid((ccz - meanz) / 5.0) * 0.95

    zv = jnp.sum((z - meanz) ** 2, axis=1, keepdims=True) / jnp.float32(_N - 1)
    ssc_ref[...] = jnp.broadcast_to(1.0 + jnp.exp(-zv / 0.1) * 0.9, (_B, 128))

    sq = x * x + y * y + z * z
    col64 = lax.broadcasted_iota(jnp.int32, (_B, 64), 1)

    def dbody(j, cd):
        sel = col_c == j
        cxj = jnp.sum(jnp.where(sel, ccx, 0.0), axis=1, keepdims=True)
        cyj = jnp.sum(jnp.where(sel, ccy, 0.0), axis=1, keepdims=True)
        czj = jnp.sum(jnp.where(sel, ccz, 0.0), axis=1, keepdims=True)
        sqc = cxj * cxj + cyj * cyj + czj * czj
        e = cxj * x + cyj * y + czj * z
        d2 = (sqc + sq) - 2.0 * e
        d = jnp.sqrt(jnp.maximum(d2, 0.0))
        cnt = jnp.sum(jnp.where(d < 0.5, 1.0, 0.0), axis=1, keepdims=True)
        return jnp.where(col64 == j, cnt, cd)

    cd = lax.fori_loop(0, 64, dbody, jnp.zeros((_B, 64), jnp.float32))
    ssd_ref[...] = 1.0 + cd / jnp.float32(_N) * 0.95

    row = lax.broadcasted_iota(jnp.int32, (_B, _C), 0)
    ci_ref[...] = ci + _N * row


def _fps_safety(xc, yc, zc):
    return pl.pallas_call(
        _fps_safety_body,
        out_shape=[
            jax.ShapeDtypeStruct((_B, _C), jnp.int32),
            jax.ShapeDtypeStruct((_B, _C), jnp.float32),
            jax.ShapeDtypeStruct((_B, 64), jnp.float32),
            jax.ShapeDtypeStruct((_B, 128), jnp.float32),
        ],
    )(xc, yc, zc)


def _sc_gather(table, idx):
    info = plsc.get_sparse_core_info()
    nc, ns = info.num_cores, info.num_subcores
    nw = nc * ns
    rows_total = _B * _C
    rpw = rows_total // nw
    mesh = plsc.VectorSubcoreMesh(core_axis_name="c", subcore_axis_name="s")

    @functools.partial(
        pl.kernel,
        mesh=mesh,
        out_type=jax.ShapeDtypeStruct((rows_total, _D), jnp.float32),
        scratch_types=[
            pltpu.VMEM((rpw,), jnp.int32),
            pltpu.VMEM((rpw, _D), jnp.float32),
            pltpu.SemaphoreType.DMA,
        ],
    )
    def gk(table_hbm, idx_hbm, out_hbm, idx_v, rows_v, sem):
        wid = lax.axis_index("s") * nc + lax.axis_index("c")
        base = wid * rpw
        pltpu.sync_copy(idx_hbm.at[pl.ds(base, rpw)], idx_v)
        pltpu.async_copy(table_hbm.at[idx_v], rows_v, sem).wait()
        pltpu.sync_copy(rows_v, out_hbm.at[pl.ds(base, rpw)])

    return gk(table, idx)


def _dotg(a, b):
    return lax.dot_general(a, b, (((1,), (1,)), ((), ())),
                           preferred_element_type=jnp.float32)


def _main_body(cf_ref, ssgT_ref, ssdT_ref, sscT_ref,
               gw1_ref, gb1_ref, gw2_ref, gb2_ref,
               cw1_ref, cb1_ref, cw2_ref, cb2_ref,
               dw1_ref, db1_ref, dw2_ref, db2_ref,
               mw1_ref, mb1_ref, mw2_ref, mb2_ref,
               lng_ref, lnb_ref,
               qa_ref, qb_ref, qs_ref,
               ka_ref, kb_ref, ks_ref,
               va_ref, vb_ref, vs_ref,
               out_ref, tok_ref):
    scales = [
        (0, 256, 16, gw1_ref, gb1_ref, gw2_ref, gb2_ref, ssgT_ref),
        (16, 128, 16, cw1_ref, cb1_ref, cw2_ref, cb2_ref, sscT_ref),
        (32, 64, 8, dw1_ref, db1_ref, dw2_ref, db2_ref, ssdT_ref),
    ]
    neg_inf = jnp.float32(-jnp.inf)
    for b in range(_B):
        for off, c_s, s_s, w1, b1, w2, b2, ss_t in scales:
            a = cf_ref[pl.ds(b * _C, c_s), :]
            h = jax.nn.relu(_dotg(a, w1[...]) + b1[...])
            p = jax.nn.sigmoid(_dotg(h, w2[...]) + b2[...])
            ss = ss_t[0:c_s, b:b + 1]
            fin = p * ss
            scores = jnp.mean(fin, axis=1, keepdims=True)
            iota_c = lax.broadcasted_iota(jnp.int32, (c_s, 1), 0)

            def pick(s, sc, b=b, off=off, c_s=c_s, iota_c=iota_c):
                mm = jnp.max(sc)
                nxt = jnp.min(jnp.where(sc == mm, iota_c, c_s))
                tok_ref[pl.ds(40 * b + off + s, 1), :] = (
                    cf_ref[pl.ds(b * _C + nxt, 1), :])
                return jnp.where(iota_c == nxt, neg_inf, sc)

            lax.fori_loop(0, s_s, pick, scores)

    tokens = tok_ref[...]
    h1 = jax.nn.relu(_dotg(tokens, mw1_ref[...]) + mb1_ref[...])
    h = _dotg(h1, mw2_ref[...]) + mb2_ref[...]
    mu = jnp.mean(h, axis=1, keepdims=True)
    var = jnp.mean((h - mu) ** 2, axis=1, keepdims=True)
    enh = (h - mu) / jnp.sqrt(var + 1e-5) * lng_ref[...] + lnb_ref[...]
    out_ref[pl.ds(0, 160), :] = enh
    heads = [(qa_ref, qb_ref, qs_ref), (ka_ref, kb_ref, ks_ref),
             (va_ref, vb_ref, vs_ref)]
    for i, (a_w, b_w, s_w) in enumerate(heads):
        t = _dotg(_dotg(enh, a_w[...]), b_w[...]) * _LORA_SCALING * s_w[...]
        out_ref[pl.ds(160 * (i + 1), 160), :] = t


def _main(cf, ssgT, ssdT, sscT, weights):
    return pl.pallas_call(
        _main_body,
        out_shape=jax.ShapeDtypeStruct((4 * _B * 40, _D), jnp.float32),
        scratch_shapes=[pltpu.VMEM((_B * 40, _D), jnp.float32)],
    )(cf, ssgT, ssdT, sscT, *weights)


def kernel(point_features, point_coords,
           g_W1, g_b1, g_W2, g_b2,
           c_W1, c_b1, c_W2, c_b2,
           d_W1, d_b1, d_W2, d_b2,
           m_W1, m_b1, m_W2, m_b2,
           ln_g, ln_b,
           q_A, q_B, q_s, k_A, k_B, k_s, v_A, v_B, v_s):
    xc = point_coords[:, :, 0]
    yc = point_coords[:, :, 1]
    zc = point_coords[:, :, 2]
    ci, ssg, ssd, ssc = _fps_safety(xc, yc, zc)
    cf = _sc_gather(point_features.reshape(_B * _N, _D), ci.reshape(_B * _C))
    weights = (
        g_W1, g_b1.reshape(1, -1), g_W2, g_b2.reshape(1, -1),
        c_W1, c_b1.reshape(1, -1), c_W2, c_b2.reshape(1, -1),
        d_W1, d_b1.reshape(1, -1), d_W2, d_b2.reshape(1, -1),
        m_W1, m_b1.reshape(1, -1), m_W2, m_b2.reshape(1, -1),
        ln_g.reshape(1, -1), ln_b.reshape(1, -1),
        q_A, q_B, q_s.reshape(1, -1),
        k_A, k_B, k_s.reshape(1, -1),
        v_A, v_B, v_s.reshape(1, -1),
    )
    out2d = _main(cf, ssg.T, ssd.T, ssc.T, weights)
    return out2d.reshape(4, _B, 40, _D)


# speedup vs baseline: 40.4314x; 1.1274x over previous
import functools

import jax
import jax.numpy as jnp
from jax import lax
from jax.experimental import pallas as pl
from jax.experimental.pallas import tpu as pltpu
from jax.experimental.pallas import tpu_sc as plsc

_B, _N, _D = 4, 4096, 768
_C = 256
_LORA_SCALING = 32.0 / 16.0


def _fps_safety_body(x_ref, y_ref, z_ref, ci_ref, ssg_ref, ssd_ref, ssc_ref):
    x = x_ref[...]
    y = y_ref[...]
    z = z_ref[...]
    sub = lax.broadcasted_iota(jnp.int32, (_B, 32, 128), 1)
    lane = lax.broadcasted_iota(jnp.int32, (_B, 32, 128), 2)
    nidx = sub * 128 + lane
    col_c = lax.broadcasted_iota(jnp.int32, (_B, _C), 1)

    def extract(v, mask):
        return jnp.sum(jnp.where(mask, v, 0.0), axis=(1, 2), keepdims=True)

    def body(i, st):
        dist, ci, ccx, ccy, ccz, last = st
        mask = nidx == last
        px = extract(x, mask)
        py = extract(y, mask)
        pz = extract(z, mask)
        ccx = jnp.where(col_c == i - 1, px[:, :, 0], ccx)
        ccy = jnp.where(col_c == i - 1, py[:, :, 0], ccy)
        ccz = jnp.where(col_c == i - 1, pz[:, :, 0], ccz)
        dx = x - px
        dy = y - py
        dz = z - pz
        d = jnp.sqrt(dx * dx + dy * dy + dz * dz)
        dist = jnp.minimum(dist, d)
        m = jnp.max(dist, axis=(1, 2), keepdims=True)
        nxt = jnp.min(jnp.where(dist == m, nidx, _N), axis=(1, 2),
                      keepdims=True)
        dist = jnp.where(nidx == nxt, 0.0, dist)
        ci = jnp.where(col_c == i, nxt[:, :, 0], ci)
        return dist, ci, ccx, ccy, ccz, nxt

    dist0 = jnp.full((_B, 32, 128), jnp.inf, jnp.float32)
    ci0 = jnp.zeros((_B, _C), jnp.int32)
    cc0 = jnp.zeros((_B, _C), jnp.float32)
    last0 = jnp.zeros((_B, 1, 1), jnp.int32)
    _, ci, ccx, ccy, ccz, last = lax.fori_loop(
        1, _C, body, (dist0, ci0, cc0, cc0, cc0, last0))
    mask = nidx == last
    px = extract(x, mask)
    py = extract(y, mask)
    pz = extract(z, mask)
    ccx = jnp.where(col_c == _C - 1, px[:, :, 0], ccx)
    ccy = jnp.where(col_c == _C - 1, py[:, :, 0], ccy)
    ccz = jnp.where(col_c == _C - 1, pz[:, :, 0], ccz)

    meanz = jnp.mean(z, axis=(1, 2), keepdims=True)
    ssg_ref[...] = 1.0 + jax.nn.sigmoid((ccz - meanz[:, :, 0]) / 5.0) * 0.95

    zv = (jnp.sum((z - meanz) ** 2, axis=(1, 2), keepdims=True)
          / jnp.float32(_N - 1))
    ssc_ref[...] = jnp.broadcast_to(
        1.0 + jnp.exp(-zv[:, :, 0] / 0.1) * 0.9, (_B, 128))

    sq = x * x + y * y + z * z
    col64 = lax.broadcasted_iota(jnp.int32, (_B, 64), 1)

    def dbody(j, cd):
        sel = col_c == j
        cxj = jnp.sum(jnp.where(sel, ccx, 0.0), axis=1, keepdims=True)
        cyj = jnp.sum(jnp.where(sel, ccy, 0.0), axis=1, keepdims=True)
        czj = jnp.sum(jnp.where(sel, ccz, 0.0), axis=1, keepdims=True)
        sqc = (cxj * cxj + cyj * cyj + czj * czj)[:, :, None]
        e = cxj[:, :, None] * x + cyj[:, :, None] * y + czj[:, :, None] * z
        d2 = (sqc + sq) - 2.0 * e
        d = jnp.sqrt(jnp.maximum(d2, 0.0))
        cnt = jnp.sum(jnp.where(d < 0.5, 1.0, 0.0), axis=(1, 2),
                      keepdims=True)
        return jnp.where(col64 == j, cnt[:, :, 0], cd)

    cd = lax.fori_loop(0, 64, dbody, jnp.zeros((_B, 64), jnp.float32))
    ssd_ref[...] = 1.0 + cd / jnp.float32(_N) * 0.95

    row = lax.broadcasted_iota(jnp.int32, (_B, _C), 0)
    ci_ref[...] = ci + _N * row


def _fps_safety(xc, yc, zc):
    return pl.pallas_call(
        _fps_safety_body,
        out_shape=[
            jax.ShapeDtypeStruct((_B, _C), jnp.int32),
            jax.ShapeDtypeStruct((_B, _C), jnp.float32),
            jax.ShapeDtypeStruct((_B, 64), jnp.float32),
            jax.ShapeDtypeStruct((_B, 128), jnp.float32),
        ],
    )(xc.reshape(_B, 32, 128), yc.reshape(_B, 32, 128),
      zc.reshape(_B, 32, 128))


def _sc_gather(table, idx):
    info = plsc.get_sparse_core_info()
    nc, ns = info.num_cores, info.num_subcores
    nw = nc * ns
    rows_total = _B * _C
    rpw = rows_total // nw
    mesh = plsc.VectorSubcoreMesh(core_axis_name="c", subcore_axis_name="s")

    @functools.partial(
        pl.kernel,
        mesh=mesh,
        out_type=jax.ShapeDtypeStruct((rows_total, _D), jnp.float32),
        scratch_types=[
            pltpu.VMEM((rpw,), jnp.int32),
            pltpu.VMEM((rpw, _D), jnp.float32),
            pltpu.SemaphoreType.DMA,
        ],
    )
    def gk(table_hbm, idx_hbm, out_hbm, idx_v, rows_v, sem):
        wid = lax.axis_index("s") * nc + lax.axis_index("c")
        base = wid * rpw
        pltpu.sync_copy(idx_hbm.at[pl.ds(base, rpw)], idx_v)
        pltpu.async_copy(table_hbm.at[idx_v], rows_v, sem).wait()
        pltpu.sync_copy(rows_v, out_hbm.at[pl.ds(base, rpw)])

    return gk(table, idx)


def _dotg(a, b):
    return lax.dot_general(a, b, (((1,), (1,)), ((), ())),
                           preferred_element_type=jnp.float32)


def _main_body(cf_ref, ssgT_ref, ssdT_ref, sscT_ref,
               gw1_ref, gb1_ref, gw2_ref, gb2_ref,
               cw1_ref, cb1_ref, cw2_ref, cb2_ref,
               dw1_ref, db1_ref, dw2_ref, db2_ref,
               mw1_ref, mb1_ref, mw2_ref, mb2_ref,
               lng_ref, lnb_ref,
               qa_ref, qb_ref, qs_ref,
               ka_ref, kb_ref, ks_ref,
               va_ref, vb_ref, vs_ref,
               out_ref, tok_ref):
    scales = [
        (0, 256, 16, gw1_ref, gb1_ref, gw2_ref, gb2_ref, ssgT_ref),
        (16, 128, 16, cw1_ref, cb1_ref, cw2_ref, cb2_ref, sscT_ref),
        (32, 64, 8, dw1_ref, db1_ref, dw2_ref, db2_ref, ssdT_ref),
    ]
    neg_inf = jnp.float32(-jnp.inf)
    for b in range(_B):
        for off, c_s, s_s, w1, b1, w2, b2, ss_t in scales:
            a = cf_ref[pl.ds(b * _C, c_s), :]
            h = jax.nn.relu(_dotg(a, w1[...]) + b1[...])
            p = jax.nn.sigmoid(_dotg(h, w2[...]) + b2[...])
            ss = ss_t[0:c_s, b:b + 1]
            fin = p * ss
            scores = jnp.mean(fin, axis=1, keepdims=True)
            iota_c = lax.broadcasted_iota(jnp.int32, (c_s, 1), 0)

            def pick(s, sc, b=b, off=off, c_s=c_s, iota_c=iota_c):
                mm = jnp.max(sc)
                nxt = jnp.min(jnp.where(sc == mm, iota_c, c_s))
                tok_ref[pl.ds(40 * b + off + s, 1), :] = (
                    cf_ref[pl.ds(b * _C + nxt, 1), :])
                return jnp.where(iota_c == nxt, neg_inf, sc)

            lax.fori_loop(0, s_s, pick, scores)

    tokens = tok_ref[...]
    h1 = jax.nn.relu(_dotg(tokens, mw1_ref[...]) + mb1_ref[...])
    h = _dotg(h1, mw2_ref[...]) + mb2_ref[...]
    mu = jnp.mean(h, axis=1, keepdims=True)
    var = jnp.mean((h - mu) ** 2, axis=1, keepdims=True)
    enh = (h - mu) / jnp.sqrt(var + 1e-5) * lng_ref[...] + lnb_ref[...]
    out_ref[pl.ds(0, 160), :] = enh
    heads = [(qa_ref, qb_ref, qs_ref), (ka_ref, kb_ref, ks_ref),
             (va_ref, vb_ref, vs_ref)]
    for i, (a_w, b_w, s_w) in enumerate(heads):
        t = _dotg(_dotg(enh, a_w[...]), b_w[...]) * _LORA_SCALING * s_w[...]
        out_ref[pl.ds(160 * (i + 1), 160), :] = t


def _main(cf, ssgT, ssdT, sscT, weights):
    return pl.pallas_call(
        _main_body,
        out_shape=jax.ShapeDtypeStruct((4 * _B * 40, _D), jnp.float32),
        scratch_shapes=[pltpu.VMEM((_B * 40, _D), jnp.float32)],
    )(cf, ssgT, ssdT, sscT, *weights)


def kernel(point_features, point_coords,
           g_W1, g_b1, g_W2, g_b2,
           c_W1, c_b1, c_W2, c_b2,
           d_W1, d_b1, d_W2, d_b2,
           m_W1, m_b1, m_W2, m_b2,
           ln_g, ln_b,
           q_A, q_B, q_s, k_A, k_B, k_s, v_A, v_B, v_s):
    xc = point_coords[:, :, 0]
    yc = point_coords[:, :, 1]
    zc = point_coords[:, :, 2]
    ci, ssg, ssd, ssc = _fps_safety(xc, yc, zc)
    cf = _sc_gather(point_features.reshape(_B * _N, _D), ci.reshape(_B * _C))
    weights = (
        g_W1, g_b1.reshape(1, -1), g_W2, g_b2.reshape(1, -1),
        c_W1, c_b1.reshape(1, -1), c_W2, c_b2.reshape(1, -1),
        d_W1, d_b1.reshape(1, -1), d_W2, d_b2.reshape(1, -1),
        m_W1, m_b1.reshape(1, -1), m_W2, m_b2.reshape(1, -1),
        ln_g.reshape(1, -1), ln_b.reshape(1, -1),
        q_A, q_B, q_s.reshape(1, -1),
        k_A, k_B, k_s.reshape(1, -1),
        v_A, v_B, v_s.reshape(1, -1),
    )
    out2d = _main(cf, ssg.T, ssd.T, ssc.T, weights)
    return out2d.reshape(4, _B, 40, _D)


# speedup vs baseline: 41.1490x; 1.0177x over previous
import functools

import jax
import jax.numpy as jnp
from jax import lax
from jax.experimental import pallas as pl
from jax.experimental.pallas import tpu as pltpu
from jax.experimental.pallas import tpu_sc as plsc

_B, _N, _D = 4, 4096, 768
_C = 256
_LORA_SCALING = 32.0 / 16.0


def _fps_safety_body(x_ref, y_ref, z_ref, ci_ref, ssg_ref, ssd_ref, ssc_ref,
                     nidx_ref):
    sub = lax.broadcasted_iota(jnp.int32, (_B, 32, 128), 1)
    lane = lax.broadcasted_iota(jnp.int32, (_B, 32, 128), 2)
    nidx_ref[...] = sub * 128 + lane
    col_c = lax.broadcasted_iota(jnp.int32, (_B, _C), 1)

    def extract(v, mask):
        return jnp.sum(jnp.where(mask, v, 0.0), axis=(1, 2), keepdims=True)

    def body(i, st):
        dist, ci, ccx, ccy, ccz, last = st
        x = x_ref[...]
        y = y_ref[...]
        z = z_ref[...]
        nidx = nidx_ref[...]
        mask = nidx == last
        dist = jnp.where(mask, 0.0, dist)
        px = extract(x, mask)
        py = extract(y, mask)
        pz = extract(z, mask)
        ccx = jnp.where(col_c == i - 1, px[:, :, 0], ccx)
        ccy = jnp.where(col_c == i - 1, py[:, :, 0], ccy)
        ccz = jnp.where(col_c == i - 1, pz[:, :, 0], ccz)
        dx = x - px
        dy = y - py
        dz = z - pz
        d = jnp.sqrt(dx * dx + dy * dy + dz * dz)
        dist = jnp.minimum(dist, d)
        m = jnp.max(dist, axis=(1, 2), keepdims=True)
        nxt = jnp.min(jnp.where(dist == m, nidx, _N), axis=(1, 2),
                      keepdims=True)
        ci = jnp.where(col_c == i, nxt[:, :, 0], ci)
        return dist, ci, ccx, ccy, ccz, nxt

    dist0 = jnp.full((_B, 32, 128), jnp.inf, jnp.float32)
    ci0 = jnp.zeros((_B, _C), jnp.int32)
    cc0 = jnp.zeros((_B, _C), jnp.float32)
    last0 = jnp.zeros((_B, 1, 1), jnp.int32)
    _, ci, ccx, ccy, ccz, last = lax.fori_loop(
        1, _C, body, (dist0, ci0, cc0, cc0, cc0, last0), unroll=2)
    x = x_ref[...]
    y = y_ref[...]
    z = z_ref[...]
    nidx = nidx_ref[...]
    mask = nidx == last
    px = extract(x, mask)
    py = extract(y, mask)
    pz = extract(z, mask)
    ccx = jnp.where(col_c == _C - 1, px[:, :, 0], ccx)
    ccy = jnp.where(col_c == _C - 1, py[:, :, 0], ccy)
    ccz = jnp.where(col_c == _C - 1, pz[:, :, 0], ccz)

    meanz = jnp.mean(z, axis=(1, 2), keepdims=True)
    ssg_ref[...] = 1.0 + jax.nn.sigmoid((ccz - meanz[:, :, 0]) / 5.0) * 0.95

    zv = (jnp.sum((z - meanz) ** 2, axis=(1, 2), keepdims=True)
          / jnp.float32(_N - 1))
    ssc_ref[...] = jnp.broadcast_to(
        1.0 + jnp.exp(-zv[:, :, 0] / 0.1) * 0.9, (_B, 128))

    col64 = lax.broadcasted_iota(jnp.int32, (_B, 64), 1)

    def dbody(j, cd):
        xd = x_ref[...]
        yd = y_ref[...]
        zd = z_ref[...]
        sel = col_c == j
        cxj = jnp.sum(jnp.where(sel, ccx, 0.0), axis=1, keepdims=True)
        cyj = jnp.sum(jnp.where(sel, ccy, 0.0), axis=1, keepdims=True)
        czj = jnp.sum(jnp.where(sel, ccz, 0.0), axis=1, keepdims=True)
        sqc = (cxj * cxj + cyj * cyj + czj * czj)[:, :, None]
        sq = xd * xd + yd * yd + zd * zd
        e = cxj[:, :, None] * xd + cyj[:, :, None] * yd + czj[:, :, None] * zd
        d2 = (sqc + sq) - 2.0 * e
        d = jnp.sqrt(jnp.maximum(d2, 0.0))
        cnt = jnp.sum(jnp.where(d < 0.5, 1.0, 0.0), axis=(1, 2),
                      keepdims=True)
        return jnp.where(col64 == j, cnt[:, :, 0], cd)

    cd = lax.fori_loop(0, 64, dbody, jnp.zeros((_B, 64), jnp.float32))
    ssd_ref[...] = 1.0 + cd / jnp.float32(_N) * 0.95

    row = lax.broadcasted_iota(jnp.int32, (_B, _C), 0)
    ci_ref[...] = ci + _N * row


def _fps_safety(xc, yc, zc):
    return pl.pallas_call(
        _fps_safety_body,
        out_shape=[
            jax.ShapeDtypeStruct((_B, _C), jnp.int32),
            jax.ShapeDtypeStruct((_B, _C), jnp.float32),
            jax.ShapeDtypeStruct((_B, 64), jnp.float32),
            jax.ShapeDtypeStruct((_B, 128), jnp.float32),
        ],
        scratch_shapes=[pltpu.VMEM((_B, 32, 128), jnp.int32)],
    )(xc.reshape(_B, 32, 128), yc.reshape(_B, 32, 128),
      zc.reshape(_B, 32, 128))


def _sc_gather(table, idx):
    info = plsc.get_sparse_core_info()
    nc, ns = info.num_cores, info.num_subcores
    nw = nc * ns
    rows_total = _B * _C
    rpw = rows_total // nw
    mesh = plsc.VectorSubcoreMesh(core_axis_name="c", subcore_axis_name="s")

    @functools.partial(
        pl.kernel,
        mesh=mesh,
        out_type=jax.ShapeDtypeStruct((rows_total, _D), jnp.float32),
        scratch_types=[
            pltpu.VMEM((rpw,), jnp.int32),
            pltpu.VMEM((rpw, _D), jnp.float32),
            pltpu.SemaphoreType.DMA,
        ],
    )
    def gk(table_hbm, idx_hbm, out_hbm, idx_v, rows_v, sem):
        wid = lax.axis_index("s") * nc + lax.axis_index("c")
        base = wid * rpw
        pltpu.sync_copy(idx_hbm.at[pl.ds(base, rpw)], idx_v)
        pltpu.async_copy(table_hbm.at[idx_v], rows_v, sem).wait()
        pltpu.sync_copy(rows_v, out_hbm.at[pl.ds(base, rpw)])

    return gk(table, idx)


def _dotg(a, b):
    return lax.dot_general(a, b, (((1,), (1,)), ((), ())),
                           preferred_element_type=jnp.float32)


def _main_body(cf_ref, ssgT_ref, ssdT_ref, sscT_ref,
               gw1_ref, gb1_ref, gw2_ref, gb2_ref,
               cw1_ref, cb1_ref, cw2_ref, cb2_ref,
               dw1_ref, db1_ref, dw2_ref, db2_ref,
               mw1_ref, mb1_ref, mw2_ref, mb2_ref,
               lng_ref, lnb_ref,
               qa_ref, qb_ref, qs_ref,
               ka_ref, kb_ref, ks_ref,
               va_ref, vb_ref, vs_ref,
               out_ref, tok_ref):
    scales = [
        (0, 256, 16, gw1_ref, gb1_ref, gw2_ref, gb2_ref, ssgT_ref),
        (16, 128, 16, cw1_ref, cb1_ref, cw2_ref, cb2_ref, sscT_ref),
        (32, 64, 8, dw1_ref, db1_ref, dw2_ref, db2_ref, ssdT_ref),
    ]
    neg_inf = jnp.float32(-jnp.inf)
    for b in range(_B):
        for off, c_s, s_s, w1, b1, w2, b2, ss_r in scales:
            a = cf_ref[pl.ds(b * _C, c_s), :]
            h = jax.nn.relu(_dotg(a, w1[...]) + b1[...])
            p = jax.nn.sigmoid(_dotg(h, w2[...]) + b2[...])
            fin = p * ss_r[0:c_s, b:b + 1]
            scores = jnp.mean(fin, axis=1, keepdims=True)
            iota_c = lax.broadcasted_iota(jnp.int32, (c_s, 1), 0)

            def pick(s, sc, b=b, off=off, c_s=c_s, iota_c=iota_c):
                mm = jnp.max(sc)
                nxt = jnp.min(jnp.where(sc == mm, iota_c, c_s))
                tok_ref[pl.ds(40 * b + off + s, 1), :] = (
                    cf_ref[pl.ds(b * _C + nxt, 1), :])
                return jnp.where(iota_c == nxt, neg_inf, sc)

            lax.fori_loop(0, s_s, pick, scores)

    tokens = tok_ref[...]
    h1 = jax.nn.relu(_dotg(tokens, mw1_ref[...]) + mb1_ref[...])
    h = _dotg(h1, mw2_ref[...]) + mb2_ref[...]
    mu = jnp.mean(h, axis=1, keepdims=True)
    var = jnp.mean((h - mu) ** 2, axis=1, keepdims=True)
    enh = (h - mu) / jnp.sqrt(var + 1e-5) * lng_ref[...] + lnb_ref[...]
    out_ref[pl.ds(0, 160), :] = enh
    heads = [(qa_ref, qb_ref, qs_ref), (ka_ref, kb_ref, ks_ref),
             (va_ref, vb_ref, vs_ref)]
    for i, (a_w, b_w, s_w) in enumerate(heads):
        t = _dotg(_dotg(enh, a_w[...]), b_w[...]) * _LORA_SCALING * s_w[...]
        out_ref[pl.ds(160 * (i + 1), 160), :] = t


def _main(cf, ssgT, ssdT, sscT, weights):
    return pl.pallas_call(
        _main_body,
        out_shape=jax.ShapeDtypeStruct((4 * _B * 40, _D), jnp.float32),
        scratch_shapes=[pltpu.VMEM((_B * 40, _D), jnp.float32)],
    )(cf, ssgT, ssdT, sscT, *weights)


def kernel(point_features, point_coords,
           g_W1, g_b1, g_W2, g_b2,
           c_W1, c_b1, c_W2, c_b2,
           d_W1, d_b1, d_W2, d_b2,
           m_W1, m_b1, m_W2, m_b2,
           ln_g, ln_b,
           q_A, q_B, q_s, k_A, k_B, k_s, v_A, v_B, v_s):
    xc = point_coords[:, :, 0]
    yc = point_coords[:, :, 1]
    zc = point_coords[:, :, 2]
    ci, ssg, ssd, ssc = _fps_safety(xc, yc, zc)
    cf = _sc_gather(point_features.reshape(_B * _N, _D), ci.reshape(_B * _C))
    weights = (
        g_W1, g_b1.reshape(1, -1), g_W2, g_b2.reshape(1, -1),
        c_W1, c_b1.reshape(1, -1), c_W2, c_b2.reshape(1, -1),
        d_W1, d_b1.reshape(1, -1), d_W2, d_b2.reshape(1, -1),
        m_W1, m_b1.reshape(1, -1), m_W2, m_b2.reshape(1, -1),
        ln_g.reshape(1, -1), ln_b.reshape(1, -1),
        q_A, q_B, q_s.reshape(1, -1),
        k_A, k_B, k_s.reshape(1, -1),
        v_A, v_B, v_s.reshape(1, -1),
    )
    out2d = _main(cf, ssg.T, ssd.T, ssc.T, weights)
    return out2d.reshape(4, _B, 40, _D)


# speedup vs baseline: 41.4487x; 1.0073x over previous
import functools

import jax
import jax.numpy as jnp
from jax import lax
from jax.experimental import pallas as pl
from jax.experimental.pallas import tpu as pltpu
from jax.experimental.pallas import tpu_sc as plsc

_B, _N, _D = 4, 4096, 768
_C = 256
_LORA_SCALING = 32.0 / 16.0


def _fps_safety_body(x_ref, y_ref, z_ref, ci_ref, ssg_ref, ssd_ref, ssc_ref,
                     nidx_ref):
    sub = lax.broadcasted_iota(jnp.int32, (_B, 32, 128), 1)
    lane = lax.broadcasted_iota(jnp.int32, (_B, 32, 128), 2)
    nidx_ref[...] = sub * 128 + lane
    col_c = lax.broadcasted_iota(jnp.int32, (_B, _C), 1)

    def extract(v, mask):
        return jnp.sum(jnp.where(mask, v, 0.0), axis=(1, 2), keepdims=True)

    def body(i, st):
        dist, ci, ccx, ccy, ccz, last = st
        x = x_ref[...]
        y = y_ref[...]
        z = z_ref[...]
        nidx = nidx_ref[...]
        mask = nidx == last
        dist = jnp.where(mask, 0.0, dist)
        px = extract(x, mask)
        py = extract(y, mask)
        pz = extract(z, mask)
        ccx = jnp.where(col_c == i - 1, px[:, :, 0], ccx)
        ccy = jnp.where(col_c == i - 1, py[:, :, 0], ccy)
        ccz = jnp.where(col_c == i - 1, pz[:, :, 0], ccz)
        dx = x - px
        dy = y - py
        dz = z - pz
        d = jnp.sqrt(dx * dx + dy * dy + dz * dz)
        dist = jnp.minimum(dist, d)
        m = jnp.max(dist, axis=(1, 2), keepdims=True)
        nxt = jnp.min(jnp.where(dist == m, nidx, _N), axis=(1, 2),
                      keepdims=True)
        ci = jnp.where(col_c == i, nxt[:, :, 0], ci)
        return dist, ci, ccx, ccy, ccz, nxt

    dist0 = jnp.full((_B, 32, 128), jnp.inf, jnp.float32)
    ci0 = jnp.zeros((_B, _C), jnp.int32)
    cc0 = jnp.zeros((_B, _C), jnp.float32)
    last0 = jnp.zeros((_B, 1, 1), jnp.int32)
    _, ci, ccx, ccy, ccz, last = lax.fori_loop(
        1, _C, body, (dist0, ci0, cc0, cc0, cc0, last0), unroll=4)
    x = x_ref[...]
    y = y_ref[...]
    z = z_ref[...]
    nidx = nidx_ref[...]
    mask = nidx == last
    px = extract(x, mask)
    py = extract(y, mask)
    pz = extract(z, mask)
    ccx = jnp.where(col_c == _C - 1, px[:, :, 0], ccx)
    ccy = jnp.where(col_c == _C - 1, py[:, :, 0], ccy)
    ccz = jnp.where(col_c == _C - 1, pz[:, :, 0], ccz)

    meanz = jnp.mean(z, axis=(1, 2), keepdims=True)
    ssg_ref[...] = 1.0 + jax.nn.sigmoid((ccz - meanz[:, :, 0]) / 5.0) * 0.95

    zv = (jnp.sum((z - meanz) ** 2, axis=(1, 2), keepdims=True)
          / jnp.float32(_N - 1))
    ssc_ref[...] = jnp.broadcast_to(
        1.0 + jnp.exp(-zv[:, :, 0] / 0.1) * 0.9, (_B, 128))

    col64 = lax.broadcasted_iota(jnp.int32, (_B, 64), 1)

    def dbody(j, cd):
        xd = x_ref[...]
        yd = y_ref[...]
        zd = z_ref[...]
        sel = col_c == j
        cxj = jnp.sum(jnp.where(sel, ccx, 0.0), axis=1, keepdims=True)
        cyj = jnp.sum(jnp.where(sel, ccy, 0.0), axis=1, keepdims=True)
        czj = jnp.sum(jnp.where(sel, ccz, 0.0), axis=1, keepdims=True)
        sqc = (cxj * cxj + cyj * cyj + czj * czj)[:, :, None]
        sq = xd * xd + yd * yd + zd * zd
        e = cxj[:, :, None] * xd + cyj[:, :, None] * yd + czj[:, :, None] * zd
        d2 = (sqc + sq) - 2.0 * e
        d = jnp.sqrt(jnp.maximum(d2, 0.0))
        cnt = jnp.sum(jnp.where(d < 0.5, 1.0, 0.0), axis=(1, 2),
                      keepdims=True)
        return jnp.where(col64 == j, cnt[:, :, 0], cd)

    cd = lax.fori_loop(0, 64, dbody, jnp.zeros((_B, 64), jnp.float32))
    ssd_ref[...] = 1.0 + cd / jnp.float32(_N) * 0.95

    row = lax.broadcasted_iota(jnp.int32, (_B, _C), 0)
    ci_ref[...] = ci + _N * row


def _fps_safety(xc, yc, zc):
    return pl.pallas_call(
        _fps_safety_body,
        out_shape=[
            jax.ShapeDtypeStruct((_B, _C), jnp.int32),
            jax.ShapeDtypeStruct((_B, _C), jnp.float32),
            jax.ShapeDtypeStruct((_B, 64), jnp.float32),
            jax.ShapeDtypeStruct((_B, 128), jnp.float32),
        ],
        scratch_shapes=[pltpu.VMEM((_B, 32, 128), jnp.int32)],
    )(xc.reshape(_B, 32, 128), yc.reshape(_B, 32, 128),
      zc.reshape(_B, 32, 128))


def _sc_gather(table, idx):
    info = plsc.get_sparse_core_info()
    nc, ns = info.num_cores, info.num_subcores
    nw = nc * ns
    rows_total = _B * _C
    rpw = rows_total // nw
    mesh = plsc.VectorSubcoreMesh(core_axis_name="c", subcore_axis_name="s")

    @functools.partial(
        pl.kernel,
        mesh=mesh,
        out_type=jax.ShapeDtypeStruct((rows_total, _D), jnp.float32),
        scratch_types=[
            pltpu.VMEM((rpw,), jnp.int32),
            pltpu.VMEM((rpw, _D), jnp.float32),
            pltpu.SemaphoreType.DMA,
        ],
    )
    def gk(table_hbm, idx_hbm, out_hbm, idx_v, rows_v, sem):
        wid = lax.axis_index("s") * nc + lax.axis_index("c")
        base = wid * rpw
        pltpu.sync_copy(idx_hbm.at[pl.ds(base, rpw)], idx_v)
        pltpu.async_copy(table_hbm.at[idx_v], rows_v, sem).wait()
        pltpu.sync_copy(rows_v, out_hbm.at[pl.ds(base, rpw)])

    return gk(table, idx)


def _dotg(a, b):
    return lax.dot_general(a, b, (((1,), (1,)), ((), ())),
                           preferred_element_type=jnp.float32)


def _main_body(cf_ref, ssgT_ref, ssdT_ref, sscT_ref,
               gw1_ref, gb1_ref, gw2_ref, gb2_ref,
               cw1_ref, cb1_ref, cw2_ref, cb2_ref,
               dw1_ref, db1_ref, dw2_ref, db2_ref,
               mw1_ref, mb1_ref, mw2_ref, mb2_ref,
               lng_ref, lnb_ref,
               qa_ref, qb_ref, qs_ref,
               ka_ref, kb_ref, ks_ref,
               va_ref, vb_ref, vs_ref,
               out_ref, tok_ref):
    scales = [
        (0, 256, 16, gw1_ref, gb1_ref, gw2_ref, gb2_ref, ssgT_ref),
        (16, 128, 16, cw1_ref, cb1_ref, cw2_ref, cb2_ref, sscT_ref),
        (32, 64, 8, dw1_ref, db1_ref, dw2_ref, db2_ref, ssdT_ref),
    ]
    neg_inf = jnp.float32(-jnp.inf)
    for b in range(_B):
        for off, c_s, s_s, w1, b1, w2, b2, ss_r in scales:
            a = cf_ref[pl.ds(b * _C, c_s), :]
            h = jax.nn.relu(_dotg(a, w1[...]) + b1[...])
            p = jax.nn.sigmoid(_dotg(h, w2[...]) + b2[...])
            fin = p * ss_r[0:c_s, b:b + 1]
            scores = jnp.mean(fin, axis=1, keepdims=True)
            iota_c = lax.broadcasted_iota(jnp.int32, (c_s, 1), 0)

            def pick(s, sc, b=b, off=off, c_s=c_s, iota_c=iota_c):
                mm = jnp.max(sc)
                nxt = jnp.min(jnp.where(sc == mm, iota_c, c_s))
                tok_ref[pl.ds(40 * b + off + s, 1), :] = (
                    cf_ref[pl.ds(b * _C + nxt, 1), :])
                return jnp.where(iota_c == nxt, neg_inf, sc)

            lax.fori_loop(0, s_s, pick, scores)

    tokens = tok_ref[...]
    h1 = jax.nn.relu(_dotg(tokens, mw1_ref[...]) + mb1_ref[...])
    h = _dotg(h1, mw2_ref[...]) + mb2_ref[...]
    mu = jnp.mean(h, axis=1, keepdims=True)
    var = jnp.mean((h - mu) ** 2, axis=1, keepdims=True)
    enh = (h - mu) / jnp.sqrt(var + 1e-5) * lng_ref[...] + lnb_ref[...]
    out_ref[pl.ds(0, 160), :] = enh
    heads = [(qa_ref, qb_ref, qs_ref), (ka_ref, kb_ref, ks_ref),
             (va_ref, vb_ref, vs_ref)]
    for i, (a_w, b_w, s_w) in enumerate(heads):
        t = _dotg(_dotg(enh, a_w[...]), b_w[...]) * _LORA_SCALING * s_w[...]
        out_ref[pl.ds(160 * (i + 1), 160), :] = t


def _main(cf, ssgT, ssdT, sscT, weights):
    return pl.pallas_call(
        _main_body,
        out_shape=jax.ShapeDtypeStruct((4 * _B * 40, _D), jnp.float32),
        scratch_shapes=[pltpu.VMEM((_B * 40, _D), jnp.float32)],
    )(cf, ssgT, ssdT, sscT, *weights)


def kernel(point_features, point_coords,
           g_W1, g_b1, g_W2, g_b2,
           c_W1, c_b1, c_W2, c_b2,
           d_W1, d_b1, d_W2, d_b2,
           m_W1, m_b1, m_W2, m_b2,
           ln_g, ln_b,
           q_A, q_B, q_s, k_A, k_B, k_s, v_A, v_B, v_s):
    xc = point_coords[:, :, 0]
    yc = point_coords[:, :, 1]
    zc = point_coords[:, :, 2]
    ci, ssg, ssd, ssc = _fps_safety(xc, yc, zc)
    cf = _sc_gather(point_features.reshape(_B * _N, _D), ci.reshape(_B * _C))
    weights = (
        g_W1, g_b1.reshape(1, -1), g_W2, g_b2.reshape(1, -1),
        c_W1, c_b1.reshape(1, -1), c_W2, c_b2.reshape(1, -1),
        d_W1, d_b1.reshape(1, -1), d_W2, d_b2.reshape(1, -1),
        m_W1, m_b1.reshape(1, -1), m_W2, m_b2.reshape(1, -1),
        ln_g.reshape(1, -1), ln_b.reshape(1, -1),
        q_A, q_B, q_s.reshape(1, -1),
        k_A, k_B, k_s.reshape(1, -1),
        v_A, v_B, v_s.reshape(1, -1),
    )
    out2d = _main(cf, ssg.T, ssd.T, ssc.T, weights)
    return out2d.reshape(4, _B, 40, _D)
